# Initial kernel scaffold; baseline (speedup 1.0000x reference)
import jax
import jax.numpy as jnp
from jax.experimental import pallas as pl


def kernel(x, wordlist):
    raise NotImplementedError("write your pallas kernel here")


# speedup vs baseline: 2.4080x; 2.4080x over previous
import functools
import math

import jax
import jax.numpy as jnp
from jax import lax
from jax.experimental import pallas as pl

SEQ = 8192
D = 512
VOCAB = 202
VPAD = 208
BLK = 512
GRID = SEQ // BLK

_LOG1E4 = math.log(10000.0)
_HALF_PI = math.pi / 2.0


def _body(x_ref, w_ref, o_ref):
    b = pl.program_id(0)

    idx = x_ref[0, 0, :]
    votes = lax.broadcasted_iota(jnp.int32, (BLK, VPAD), 1)
    onehot = (idx[:, None] == votes).astype(jnp.bfloat16)
    w = w_ref[...]
    hi = w.astype(jnp.bfloat16)
    lo = (w - hi.astype(jnp.float32)).astype(jnp.bfloat16)
    g = jnp.dot(onehot, hi, preferred_element_type=jnp.float32)
    g = g + jnp.dot(onehot, lo, preferred_element_type=jnp.float32)

    c = lax.broadcasted_iota(jnp.int32, (BLK, D), 1)
    jj = (c >> 1).astype(jnp.float32)
    inv_denom = jnp.exp(jj * (-2.0 * _LOG1E4 / D))
    i = (b * BLK + lax.broadcasted_iota(jnp.int32, (BLK, D), 0)).astype(
        jnp.float32
    )
    ang = i * inv_denom + (c & 1).astype(jnp.float32) * _HALF_PI
    pm = jnp.where(c >= D - 2, 0.0, jnp.sin(ang))

    o_ref[...] = g + pm


@functools.partial(jax.jit)
def kernel(x, wordlist):
    xb = x.reshape(GRID, 1, BLK)
    wp = jnp.pad(wordlist, ((0, VPAD - VOCAB), (0, 0)))
    return pl.pallas_call(
        _body,
        grid=(GRID,),
        in_specs=[
            pl.BlockSpec((1, 1, BLK), lambda b: (b, 0, 0)),
            pl.BlockSpec((VPAD, D), lambda b: (0, 0)),
        ],
        out_specs=pl.BlockSpec((BLK, D), lambda b: (b, 0)),
        out_shape=jax.ShapeDtypeStruct((SEQ, D), jnp.float32),
    )(xb, wp)


# speedup vs baseline: 9.1291x; 3.7912x over previous
import functools
import math

import jax
import jax.numpy as jnp
from jax import lax
from jax.experimental import pallas as pl
from jax.experimental.pallas import tpu as pltpu

SEQ = 8192
D = 512
VOCAB = 202
VPAD = 208
BLK = 512
GRID = SEQ // BLK
NH = BLK // 64

_NEG2LOG1E4_D = -2.0 * math.log(10000.0) / D


def _body(x_ref, w_ref, o_ref, sl_ref, cl_ref):
    b = pl.program_id(0)

    @pl.when(b == 0)
    def _init():
        c = lax.broadcasted_iota(jnp.int32, (64, D), 1)
        inv = jnp.exp((c >> 1).astype(jnp.float32) * _NEG2LOG1E4_D)
        l = lax.broadcasted_iota(jnp.int32, (64, D), 0).astype(jnp.float32)
        ang = l * inv
        sl_ref[...] = jnp.sin(ang)
        cl_ref[...] = jnp.cos(ang)

    idx = x_ref[0, 0, :]
    votes = lax.broadcasted_iota(jnp.int32, (BLK, VPAD), 1)
    onehot = (idx[:, None] == votes).astype(jnp.bfloat16)
    w = w_ref[...]
    hi = w.astype(jnp.bfloat16)
    lo = (w - hi.astype(jnp.float32)).astype(jnp.bfloat16)
    g = jnp.dot(onehot, hi, preferred_element_type=jnp.float32)
    g = g + jnp.dot(onehot, lo, preferred_element_type=jnp.float32)

    ch = lax.broadcasted_iota(jnp.int32, (NH, D), 1)
    invh = jnp.exp((ch >> 1).astype(jnp.float32) * _NEG2LOG1E4_D)
    hh = lax.broadcasted_iota(jnp.int32, (NH, D), 0)
    base = (b * BLK + hh * 64).astype(jnp.float32)
    ang_h = base * invh
    sh = jnp.sin(ang_h)
    chc = jnp.cos(ang_h)
    even = (ch & 1) == 0
    live = ch < D - 2
    u = jnp.where(even & live, sh, jnp.where(live, chc, 0.0))
    v = jnp.where(even & live, chc, jnp.where(live, -sh, 0.0))

    cl = cl_ref[...]
    sl = sl_ref[...]
    for h in range(NH):
        pm = u[h : h + 1, :] * cl + v[h : h + 1, :] * sl
        o_ref[h * 64 : (h + 1) * 64, :] = g[h * 64 : (h + 1) * 64, :] + pm


@functools.partial(jax.jit)
def kernel(x, wordlist):
    xb = x.reshape(GRID, 1, BLK)
    wp = jnp.pad(wordlist, ((0, VPAD - VOCAB), (0, 0)))
    return pl.pallas_call(
        _body,
        grid=(GRID,),
        in_specs=[
            pl.BlockSpec((1, 1, BLK), lambda b: (b, 0, 0)),
            pl.BlockSpec((VPAD, D), lambda b: (0, 0)),
        ],
        out_specs=pl.BlockSpec((BLK, D), lambda b: (b, 0)),
        out_shape=jax.ShapeDtypeStruct((SEQ, D), jnp.float32),
        scratch_shapes=[
            pltpu.VMEM((64, D), jnp.float32),
            pltpu.VMEM((64, D), jnp.float32),
        ],
    )(xb, wp)
